# baseline (speedup 1.0000x reference)
import jax
import jax.numpy as jnp
from jax.experimental import pallas as pl
from jax.experimental.pallas import tpu as pltpu

_BI = 400


def _row_block(p, i, nb):
    return i + p * (nb - 1 - 2 * i)


def _fused_body(adj_ref, x_ref, w1_ref, b1_ref, w2_ref, b2_ref, o_ref, h_ref,
                xb_ref):
    p = pl.program_id(0)
    i = _row_block(p, pl.program_id(1), pl.num_programs(1))
    a = adj_ref[...].astype(jnp.bfloat16)

    @pl.when((p == 0) & (pl.program_id(1) == 0))
    def _cast_x_once():
        xb_ref[...] = x_ref[...].astype(jnp.bfloat16)

    @pl.when(p == 0)
    def _layer1():
        t = jnp.dot(a, xb_ref[...], preferred_element_type=jnp.float32)
        o = jnp.dot(t, w1_ref[...], preferred_element_type=jnp.float32)
        o = jnp.maximum(o + b1_ref[...], 0.0)
        h_ref[pl.ds(i * _BI, _BI), :] = o.astype(jnp.bfloat16)

    @pl.when(p == 1)
    def _layer2():
        t = jnp.dot(a, h_ref[...], preferred_element_type=jnp.float32)
        o_ref[...] = jnp.dot(t, w2_ref[...],
                             preferred_element_type=jnp.float32) + b2_ref[...]


def kernel(adj, x, W1, b1, W2, b2):
    n, _ = adj.shape
    d = W2.shape[1]
    return pl.pallas_call(
        _fused_body,
        grid=(2, pl.cdiv(n, _BI)),
        in_specs=[
            pl.BlockSpec((_BI, n),
                         lambda p, i: (_row_block(p, i, pl.num_programs(1)), 0)),
            pl.BlockSpec((n, x.shape[1]), lambda p, i: (0, 0)),
            pl.BlockSpec((x.shape[1], W1.shape[1]), lambda p, i: (0, 0)),
            pl.BlockSpec((1, W1.shape[1]), lambda p, i: (0, 0)),
            pl.BlockSpec((W1.shape[1], d), lambda p, i: (0, 0)),
            pl.BlockSpec((1, d), lambda p, i: (0, 0)),
        ],
        out_specs=pl.BlockSpec(
            (_BI, d), lambda p, i: (p * _row_block(p, i, pl.num_programs(1)), 0)),
        out_shape=jax.ShapeDtypeStruct((n, d), jnp.float32),
        scratch_shapes=[pltpu.VMEM((n, W1.shape[1]), jnp.bfloat16),
                        pltpu.VMEM((n, x.shape[1]), jnp.bfloat16)],
    )(adj, x, W1, b1.reshape(1, -1), W2, b2.reshape(1, -1))


# speedup vs baseline: 1.0036x; 1.0036x over previous
import jax
import jax.numpy as jnp
from jax.experimental import pallas as pl
from jax.experimental.pallas import tpu as pltpu

_BI = 400


def _row_block(p, i, nb):
    return i + p * (nb - 1 - 2 * i)


def _fused_body(adj_ref, x_ref, w1_ref, b1_ref, w2_ref, b2_ref, o_ref, h_ref):
    p = pl.program_id(0)
    i = _row_block(p, pl.program_id(1), pl.num_programs(1))
    a = adj_ref[...].astype(jnp.bfloat16)

    @pl.when(p == 0)
    def _layer1():
        t = jnp.dot(a, x_ref[...].astype(jnp.bfloat16),
                    preferred_element_type=jnp.float32)
        o = jnp.dot(t, w1_ref[...], preferred_element_type=jnp.float32)
        o = jnp.maximum(o + b1_ref[...], 0.0)
        h_ref[pl.ds(i * _BI, _BI), :] = o.astype(jnp.bfloat16)

    @pl.when(p == 1)
    def _layer2():
        t = jnp.dot(a, h_ref[...], preferred_element_type=jnp.float32)
        o_ref[...] = jnp.dot(t, w2_ref[...],
                             preferred_element_type=jnp.float32) + b2_ref[...]


def kernel(adj, x, W1, b1, W2, b2):
    n, _ = adj.shape
    d = W2.shape[1]
    return pl.pallas_call(
        _fused_body,
        grid=(2, pl.cdiv(n, _BI)),
        in_specs=[
            pl.BlockSpec((_BI, n),
                         lambda p, i: (_row_block(p, i, pl.num_programs(1)), 0)),
            pl.BlockSpec((n, x.shape[1]), lambda p, i: (0, 0)),
            pl.BlockSpec((x.shape[1], W1.shape[1]), lambda p, i: (0, 0)),
            pl.BlockSpec((1, W1.shape[1]), lambda p, i: (0, 0)),
            pl.BlockSpec((W1.shape[1], d), lambda p, i: (0, 0)),
            pl.BlockSpec((1, d), lambda p, i: (0, 0)),
        ],
        out_specs=pl.BlockSpec(
            (_BI, d), lambda p, i: (p * _row_block(p, i, pl.num_programs(1)), 0)),
        out_shape=jax.ShapeDtypeStruct((n, d), jnp.float32),
        scratch_shapes=[pltpu.VMEM((n, W1.shape[1]), jnp.bfloat16)],
    )(adj, x, W1, b1.reshape(1, -1), W2, b2.reshape(1, -1))
